# Initial kernel scaffold; baseline (speedup 1.0000x reference)
import jax
import jax.numpy as jnp
from jax.experimental import pallas as pl


def kernel(boxes, scores):
    raise NotImplementedError("write your pallas kernel here")


# speedup vs baseline: 19.0599x; 19.0599x over previous
import functools

import jax
import jax.numpy as jnp
from jax.experimental import pallas as pl

CONF_THRES = 0.25
IOU_THRES = 0.45
MAX_OUT = 100

_N = 20000
_ROWS = 160
_COLS = 128
_NPAD = _ROWS * _COLS

_NEG_INF = float("-inf")


def _nms_body(x1_ref, y1_ref, x2_ref, y2_ref, s_ref, out_ref):
    x1 = x1_ref[...]
    y1 = y1_ref[...]
    x2 = x2_ref[...]
    y2 = y2_ref[...]
    s0 = s_ref[...]

    row_i = jax.lax.broadcasted_iota(jnp.int32, (_ROWS, _COLS), 0)
    col_i = jax.lax.broadcasted_iota(jnp.int32, (_ROWS, _COLS), 1)
    iota = row_i * _COLS + col_i

    area = jnp.maximum(0.0, x2 - x1) * jnp.maximum(0.0, y2 - y1)
    s_work = jnp.where(s0 > CONF_THRES, s0, _NEG_INF)

    lastm = iota == (_N - 1)
    fx1 = jnp.sum(jnp.where(lastm, x1, 0.0))
    fy1 = jnp.sum(jnp.where(lastm, y1, 0.0))
    fx2 = jnp.sum(jnp.where(lastm, x2, 0.0))
    fy2 = jnp.sum(jnp.where(lastm, y2, 0.0))
    fs = jnp.sum(jnp.where(lastm, s0, 0.0))

    lane = jax.lax.broadcasted_iota(jnp.int32, (1, _COLS), 1)

    def it(k, s):
        m = jnp.max(s)
        has = m > _NEG_INF
        i = jnp.min(jnp.where(s == m, iota, _NPAD))
        im = iota == i
        bx1 = jnp.sum(jnp.where(im, x1, 0.0))
        by1 = jnp.sum(jnp.where(im, y1, 0.0))
        bx2 = jnp.sum(jnp.where(im, x2, 0.0))
        by2 = jnp.sum(jnp.where(im, y2, 0.0))
        ba = jnp.sum(jnp.where(im, area, 0.0))

        ox1 = jnp.where(has, bx1, fx1)
        oy1 = jnp.where(has, by1, fy1)
        ox2 = jnp.where(has, bx2, fx2)
        oy2 = jnp.where(has, by2, fy2)
        osc = jnp.where(has, m, fs)
        row = jnp.where(
            lane == 0, ox1,
            jnp.where(lane == 1, oy1,
                      jnp.where(lane == 2, ox2,
                                jnp.where(lane == 3, oy2,
                                          jnp.where(lane == 4, osc, 0.0)))))
        out_ref[pl.ds(k, 1), :] = row

        xx1 = jnp.maximum(bx1, x1)
        yy1 = jnp.maximum(by1, y1)
        xx2 = jnp.minimum(bx2, x2)
        yy2 = jnp.minimum(by2, y2)
        inter = jnp.maximum(0.0, xx2 - xx1) * jnp.maximum(0.0, yy2 - yy1)
        iou = inter / (ba + area - inter + 1e-9)
        s = jnp.where(has & ((iou > IOU_THRES) | im), _NEG_INF, s)
        return s

    jax.lax.fori_loop(0, MAX_OUT, it, s_work)


@functools.partial(jax.jit)
def kernel(boxes, scores):
    pad = _NPAD - _N
    x1 = jnp.pad(boxes[:, 0], (0, pad)).reshape(_ROWS, _COLS)
    y1 = jnp.pad(boxes[:, 1], (0, pad)).reshape(_ROWS, _COLS)
    x2 = jnp.pad(boxes[:, 2], (0, pad)).reshape(_ROWS, _COLS)
    y2 = jnp.pad(boxes[:, 3], (0, pad)).reshape(_ROWS, _COLS)
    s = jnp.pad(scores, (0, pad)).reshape(_ROWS, _COLS)

    out = pl.pallas_call(
        _nms_body,
        out_shape=jax.ShapeDtypeStruct((MAX_OUT, _COLS), jnp.float32),
    )(x1, y1, x2, y2, s)

    return out[:, :4], out[:, 4]
